# baseline (speedup 1.0000x reference)
import functools

import jax
import jax.numpy as jnp
from jax import lax
from jax.experimental import pallas as pl
from jax.experimental.pallas import tpu as pltpu
from jax.experimental.pallas import tpu_sc as plsc

N = 50000
D = 128
H = 64
OUT = 128
G = 512
E = 400000

BS = 2000
NBLK = N // BS

TICK = 128
ETICKS = 3200
EPAD = ETICKS * TICK - E
TRASH = N
ACC_ROWS = N + 48
CNT_ROWS = 3200
SUB_T = ETICKS // 16
SB = 8
N_EPS = 1e-5

_SC_MESH = dict(
    mesh=plsc.VectorSubcoreMesh(core_axis_name="c", subcore_axis_name="s"),
    compiler_params=pltpu.CompilerParams(use_tc_tiling_on_sc=False),
)


def _tc_prep_body(xm, xc, wl1a, wl1b, wr1a, wr1b, bl1a, bl1b,
                  y1m, y1c, r1m, r1c):
    xmv = xm[...]
    xcv = xc[...]
    y1m[...] = xcv @ wl1a[0]
    y1c[...] = xmv @ wl1b[0]
    r1m[...] = xmv @ wr1a[0] + bl1a[0]
    r1c[...] = xcv @ wr1b[0] + bl1b[0]


def _halves(w):
    return w.reshape(w.shape[0], 2, 32).transpose(1, 0, 2)


def _tc_prep(x_main, x_ctx, Wl1_c2m, Wl1_m2c, Wr1_c2m, Wr1_m2c, bl1_c2m, bl1_m2c):
    spec_x = pl.BlockSpec((BS, D), lambda j, i: (i, 0))
    spec_w = pl.BlockSpec((1, D, 32), lambda j, i: (j, 0, 0))
    spec_b = pl.BlockSpec((1, 1, 32), lambda j, i: (j, 0, 0))
    spec_o = pl.BlockSpec((BS, 32), lambda j, i: (j * NBLK + i, 0))
    oshape = jax.ShapeDtypeStruct((2 * N, 32), jnp.float32)
    return pl.pallas_call(
        _tc_prep_body,
        grid=(2, NBLK),
        in_specs=[spec_x, spec_x, spec_w, spec_w, spec_w, spec_w, spec_b, spec_b],
        out_specs=[spec_o, spec_o, spec_o, spec_o],
        out_shape=[oshape, oshape, oshape, oshape],
    )(x_main, x_ctx, _halves(Wl1_c2m), _halves(Wl1_m2c), _halves(Wr1_c2m),
      _halves(Wr1_m2c), bl1_c2m.reshape(2, 1, 32), bl1_m2c.reshape(2, 1, 32))


AROWS = 50048


SSB = 2
NSB = SUB_T // SSB


def _seg_common(y, src, dst, out, bufs, acc, c, s):
    rows0 = bufs[0][4]
    zeros16 = jnp.zeros((16,), jnp.float32)
    cN = c * N

    def _fire_idx(sb, b):
        srcb, dstb, _, _, _, _, _, isem = bufs[b]
        tb = s * SUB_T + sb * SSB
        pltpu.async_copy(src.at[pl.ds(tb, SSB)], srcb, isem)
        pltpu.async_copy(dst.at[pl.ds(tb, SSB)], dstb, isem)

    def _stage(sb, b, drain_scatter):
        srcb, dstb, idxb, ldstb, rowsb, gsem, ssem, isem = bufs[b]
        tb = s * SUB_T + sb * SSB
        pltpu.make_async_copy(src.at[pl.ds(tb, SSB)], srcb, isem).wait()
        pltpu.make_async_copy(dst.at[pl.ds(tb, SSB)], dstb, isem).wait()
        if drain_scatter:
            for r in range(SSB):
                pltpu.make_async_copy(rowsb.at[pl.ds(r * TICK, TICK)],
                                      acc.at[pl.ds(0, TICK)], ssem).wait()
        for r in range(SSB):
            for t in range(8):
                sl = pl.ds(t * 16, 16)
                idxb[r, sl] = srcb[r, sl] + cN
                ldstb[r, sl] = dstb[r, sl]

        @pl.when(sb + 2 < NSB)
        def _():
            _fire_idx(sb + 2, b)
        handles = []
        for r in range(SSB):
            handles.append(pltpu.async_copy(
                y.at[idxb.at[r]], rowsb.at[pl.ds(r * TICK, TICK)], gsem))
        for h in handles:
            h.wait()
        for r in range(SSB):
            pltpu.async_copy(rowsb.at[pl.ds(r * TICK, TICK)],
                             acc.at[ldstb.at[r]], ssem, add=True)

    def _drain(b):
        _, _, _, _, rowsb, _, ssem, _ = bufs[b]
        for r in range(SSB):
            pltpu.make_async_copy(rowsb.at[pl.ds(r * TICK, TICK)],
                                  acc.at[pl.ds(0, TICK)], ssem).wait()

    def _zero(r, carry):
        rows0[r, pl.ds(0, 16)] = zeros16
        rows0[r, pl.ds(16, 16)] = zeros16
        return carry
    lax.fori_loop(0, SSB * TICK, _zero, 0)
    for k in range(12):
        pltpu.sync_copy(rows0, acc.at[pl.ds(s * 3128 + k * SSB * TICK, SSB * TICK)])
    pltpu.sync_copy(rows0.at[pl.ds(0, 56)],
                    acc.at[pl.ds(s * 3128 + 12 * SSB * TICK, 56)])
    _fire_idx(0, 0)
    _fire_idx(1, 1)
    plsc.subcore_barrier()

    _stage(0, 0, drain_scatter=False)
    _stage(1, 1, drain_scatter=False)

    def body(k, carry):
        _stage(2 + 2 * k, 0, drain_scatter=True)
        _stage(3 + 2 * k, 1, drain_scatter=True)
        return carry
    lax.fori_loop(0, (NSB - 2) // 2, body, 0)
    _drain(0)
    _drain(1)
    plsc.subcore_barrier()

    @pl.when(s < 15)
    def _():
        pltpu.sync_copy(acc.at[pl.ds(s * 3128, 3128)],
                        out.at[pl.ds(c * N + s * 3128, 3128)])

    @pl.when(s == 15)
    def _():
        pltpu.sync_copy(acc.at[pl.ds(15 * 3128, 3080)],
                        out.at[pl.ds(c * N + 15 * 3128, 3080)])


def _sc_segsum_body(y, src, dst, out,
                    srcb0, dstb0, idxb0, ldst0, rows0,
                    srcb1, dstb1, idxb1, ldst1, rows1,
                    acc, gsem0, gsem1, ssem0, ssem1, isem0, isem1):
    c = lax.axis_index("c")
    s = lax.axis_index("s")
    bufs = ((srcb0, dstb0, idxb0, ldst0, rows0, gsem0, ssem0, isem0),
            (srcb1, dstb1, idxb1, ldst1, rows1, gsem1, ssem1, isem1))
    _seg_common(y, src, dst, out, bufs, acc, c, s)


def _sc_l1_body(ym, yc, srcm, dstm, srcc, dstc, aggm, aggc, cnts,
                srcb0, dstb0, idxb0, ldst0, rows0,
                srcb1, dstb1, idxb1, ldst1, rows1,
                acc, gsem0, gsem1, ssem0, ssem1, isem0, isem1):
    c = lax.axis_index("c")
    s = lax.axis_index("s")
    zeros16 = jnp.zeros((16,), jnp.float32)
    ones16 = jnp.ones((16,), jnp.float32)
    bufs = ((srcb0, dstb0, idxb0, ldst0, rows0, gsem0, ssem0, isem0),
            (srcb1, dstb1, idxb1, ldst1, rows1, gsem1, ssem1, isem1))

    def _fill(r, carry):
        rows0[r, pl.ds(0, 16)] = zeros16
        rows0[r, pl.ds(16, 16)] = zeros16
        rows1[r, pl.ds(0, 16)] = ones16
        rows1[r, pl.ds(16, 16)] = ones16
        return carry
    lax.fori_loop(0, SSB * TICK, _fill, 0)
    for k in range(12):
        pltpu.sync_copy(rows0, acc.at[pl.ds(s * 3128 + k * SSB * TICK, SSB * TICK)])
    pltpu.sync_copy(rows0.at[pl.ds(0, 56)],
                    acc.at[pl.ds(s * 3128 + 12 * SSB * TICK, 56)])
    plsc.subcore_barrier()

    def _histogram(dref):
        def body(it, carry):
            tb = s * SUB_T + it * SSB
            pltpu.sync_copy(dref.at[pl.ds(tb, SSB)], srcb0)
            for r in range(SSB):
                pltpu.sync_copy(rows1.at[pl.ds(0, TICK)],
                                acc.at[srcb0.at[r]], add=True)
            return carry
        lax.fori_loop(0, NSB, body, 0)

    @pl.when(c == 0)
    def _():
        _histogram(dstm)

    @pl.when(c == 1)
    def _():
        _histogram(dstc)

    plsc.subcore_barrier()
    pltpu.sync_copy(acc.at[pl.ds(s * 3128, 3128)],
                    cnts.at[c, pl.ds(s * 3128, 3128)])
    plsc.subcore_barrier()

    _seg_common(ym, srcm, dstm, aggm, bufs, acc, c, s)
    plsc.subcore_barrier()
    _seg_common(yc, srcc, dstc, aggc, bufs, acc, c, s)


def _seg_bufs():
    return [
        pltpu.VMEM((SSB, TICK), jnp.int32),
        pltpu.VMEM((SSB, TICK), jnp.int32),
        pltpu.VMEM((SSB, TICK), jnp.int32),
        pltpu.VMEM((SSB, TICK), jnp.int32),
        pltpu.VMEM((SSB * TICK, 32), jnp.float32),
    ]


_SEG_SCRATCH = _seg_bufs() + _seg_bufs() + [
    pltpu.VMEM_SHARED((AROWS, 32), jnp.float32),
    pltpu.SemaphoreType.DMA,
    pltpu.SemaphoreType.DMA,
    pltpu.SemaphoreType.DMA,
    pltpu.SemaphoreType.DMA,
    pltpu.SemaphoreType.DMA,
    pltpu.SemaphoreType.DMA,
]

_sc_segsum = functools.partial(
    pl.kernel,
    out_type=jax.ShapeDtypeStruct((2 * N, 32), jnp.float32),
    scratch_types=_SEG_SCRATCH,
    **_SC_MESH,
)(_sc_segsum_body)

_sc_l1 = functools.partial(
    pl.kernel,
    out_type=(jax.ShapeDtypeStruct((2 * N, 32), jnp.float32),
              jax.ShapeDtypeStruct((2 * N, 32), jnp.float32),
              jax.ShapeDtypeStruct((2, AROWS, 32), jnp.float32)),
    scratch_types=_SEG_SCRATCH,
    **_SC_MESH,
)(_sc_l1_body)


def _sc_pool_body(z2, batch, out, bbuf, btail, rowsb, pacc):
    c = lax.axis_index("c")
    s = lax.axis_index("s")
    w = c * 16 + s
    zeros16 = jnp.zeros((16,), jnp.float32)

    def _zero(r, carry):
        for q in range(8):
            rowsb[r, pl.ds(q * 16, 16)] = zeros16
        return carry
    lax.fori_loop(0, 32, _zero, 0)
    pltpu.sync_copy(rowsb.at[pl.ds(0, 32)], pacc.at[pl.ds(s * 32, 32)])

    @pl.when(s == 0)
    def _():
        pltpu.sync_copy(rowsb.at[pl.ds(0, 32)], pacc.at[pl.ds(G, 32)])
    plsc.subcore_barrier()

    off = 12 * w + jnp.minimum(w, 7)
    n_w = jnp.where(w == 31, 11, jnp.where(w < 7, 13, 12))

    def body(k, carry):
        tk = off + k
        pltpu.sync_copy(z2.at[pl.ds(tk * 128, 128)], rowsb)
        pltpu.sync_copy(batch.at[pl.ds(tk * 128, 128)], bbuf)
        pltpu.sync_copy(rowsb, pacc.at[bbuf], add=True)
        return carry
    lax.fori_loop(0, n_w, body, 0)

    @pl.when(w == 31)
    def _():
        pltpu.sync_copy(z2.at[pl.ds(390 * 128, 80)], rowsb.at[pl.ds(0, 80)])
        pltpu.sync_copy(batch.at[pl.ds(390 * 128, 80)], btail)
        pltpu.sync_copy(rowsb.at[pl.ds(0, 80)], pacc.at[btail], add=True)

    plsc.subcore_barrier()
    pltpu.sync_copy(pacc.at[pl.ds(s * 32, 32)], out.at[c, pl.ds(s * 32, 32)])


_sc_pool = functools.partial(
    pl.kernel,
    out_type=jax.ShapeDtypeStruct((2, G, OUT), jnp.float32),
    scratch_types=[
        pltpu.VMEM((128,), jnp.int32),
        pltpu.VMEM((80,), jnp.int32),
        pltpu.VMEM((128, OUT), jnp.float32),
        pltpu.VMEM_SHARED((G + 32, OUT), jnp.float32),
    ],
    **_SC_MESH,
)(_sc_pool_body)


def _tc_b1_main_body(alo, ahi, rlo, rhi, cnt, z_ref, st_ref):
    i = pl.program_id(0)
    cdiv = jnp.maximum(cnt[...], 1.0)
    a = jnp.concatenate([alo[...], ahi[...]], axis=1) / cdiv
    r = jnp.concatenate([rlo[...], rhi[...]], axis=1)
    z = jnp.maximum(a + r, 0.0)
    z_ref[...] = z

    @pl.when(i == 0)
    def _():
        st_ref[...] = jnp.zeros_like(st_ref)
    st_ref[...] += jnp.stack([jnp.sum(z, axis=0), jnp.sum(z * z, axis=0)])


def _tc_b1_main(agg, r2, cnt):
    return pl.pallas_call(
        _tc_b1_main_body,
        grid=(NBLK,),
        in_specs=[
            pl.BlockSpec((BS, 32), lambda i: (i, 0)),
            pl.BlockSpec((BS, 32), lambda i: (NBLK + i, 0)),
            pl.BlockSpec((BS, 32), lambda i: (i, 0)),
            pl.BlockSpec((BS, 32), lambda i: (NBLK + i, 0)),
            pl.BlockSpec((BS, 1), lambda i: (i, 0)),
        ],
        out_specs=[
            pl.BlockSpec((BS, H), lambda i: (i, 0)),
            pl.BlockSpec((2, H), lambda i: (0, 0)),
        ],
        out_shape=[
            jax.ShapeDtypeStruct((N, H), jnp.float32),
            jax.ShapeDtypeStruct((2, H), jnp.float32),
        ],
    )(agg, agg, r2, r2, cnt)


def _tc_b1_ctx_body(a_ref, r_ref, cnt, z_ref, st_ref):
    i = pl.program_id(1)
    cdiv = jnp.maximum(cnt[...], 1.0)
    z = jnp.maximum(a_ref[...] / cdiv + r_ref[...], 0.0)
    z_ref[...] = z

    @pl.when(i == 0)
    def _():
        st_ref[...] = jnp.zeros_like(st_ref)
    st_ref[...] += jnp.stack([jnp.sum(z, axis=0), jnp.sum(z * z, axis=0)])[None]


def _tc_b1_ctx(agg, r2, cnt):
    spec_half = pl.BlockSpec((BS, 32), lambda j, i: (j * NBLK + i, 0))
    return pl.pallas_call(
        _tc_b1_ctx_body,
        grid=(2, NBLK),
        in_specs=[
            spec_half,
            spec_half,
            pl.BlockSpec((BS, 1), lambda j, i: (i, 0)),
        ],
        out_specs=[
            spec_half,
            pl.BlockSpec((1, 2, 32), lambda j, i: (j, 0, 0)),
        ],
        out_shape=[
            jax.ShapeDtypeStruct((2 * N, 32), jnp.float32),
            jax.ShapeDtypeStruct((2, 2, 32), jnp.float32),
        ],
    )(agg, r2, cnt)


def _tc_c1_body(alo, ahi, cnt, st_ref, g_ref, b_ref, w_ref,
                z1m_ref, stm_ref, wr_ref, bl_ref, batch_ref,
                z2_ref, st2_ref, cg_ref):
    i = pl.program_id(0)
    st = st_ref[...]
    mu = st[0:1, :] * (1.0 / N)
    var = st[1:2, :] * (1.0 / N) - mu * mu
    sca = g_ref[...] * lax.rsqrt(var + N_EPS)
    craw = cnt[...]
    cdiv = jnp.maximum(craw, 1.0)
    a = jnp.concatenate([alo[...], ahi[...]], axis=1) / cdiv
    a2 = ((a - mu) * sca + b_ref[...]) * (craw > 0.0).astype(jnp.float32)
    stm = stm_ref[...]
    mum = stm[0:1, :] * (1.0 / N)
    varm = stm[1:2, :] * (1.0 / N) - mum * mum
    scam = g_ref[...] * lax.rsqrt(varm + N_EPS)
    h = (z1m_ref[...] - mum) * scam + b_ref[...]
    z = jnp.maximum(a2 @ w_ref[...] + h @ wr_ref[...] + bl_ref[...], 0.0)
    z2_ref[...] = z

    @pl.when(i == 0)
    def _():
        st2_ref[...] = jnp.zeros_like(st2_ref)
        cg_ref[...] = jnp.zeros_like(cg_ref)
    st2_ref[...] += jnp.stack([jnp.sum(z, axis=0), jnp.sum(z * z, axis=0)])
    onehot = (batch_ref[...] ==
              lax.broadcasted_iota(jnp.int32, (BS, G), 1)).astype(jnp.float32)
    cg_ref[...] += jnp.sum(onehot, axis=0)[None, :]


def _tc_c1(agg2, cnt, st1c, g1, b1, Wl2, z1m, st1m, Wr2, bl2, batch):
    return pl.pallas_call(
        _tc_c1_body,
        grid=(NBLK,),
        in_specs=[
            pl.BlockSpec((BS, 32), lambda i: (i, 0)),
            pl.BlockSpec((BS, 32), lambda i: (NBLK + i, 0)),
            pl.BlockSpec((BS, 1), lambda i: (i, 0)),
            pl.BlockSpec((2, H), lambda i: (0, 0)),
            pl.BlockSpec((1, H), lambda i: (0, 0)),
            pl.BlockSpec((1, H), lambda i: (0, 0)),
            pl.BlockSpec((H, OUT), lambda i: (0, 0)),
            pl.BlockSpec((BS, H), lambda i: (i, 0)),
            pl.BlockSpec((2, H), lambda i: (0, 0)),
            pl.BlockSpec((H, OUT), lambda i: (0, 0)),
            pl.BlockSpec((1, OUT), lambda i: (0, 0)),
            pl.BlockSpec((BS, 1), lambda i: (i, 0)),
        ],
        out_specs=[
            pl.BlockSpec((BS, OUT), lambda i: (i, 0)),
            pl.BlockSpec((2, OUT), lambda i: (0, 0)),
            pl.BlockSpec((1, G), lambda i: (0, 0)),
        ],
        out_shape=[
            jax.ShapeDtypeStruct((N, OUT), jnp.float32),
            jax.ShapeDtypeStruct((2, OUT), jnp.float32),
            jax.ShapeDtypeStruct((1, G), jnp.float32),
        ],
    )(agg2, agg2, cnt, st1c, g1.reshape(1, H), b1.reshape(1, H), Wl2,
      z1m, st1m, Wr2, bl2.reshape(1, OUT), batch)


def _tc_d_body(pool_ref, cg_ref, st2_ref, g_ref, b_ref, wp_ref, bp_ref, out_ref):
    st = st2_ref[...]
    mu = st[0:1, :] * (1.0 / N)
    var = st[1:2, :] * (1.0 / N) - mu * mu
    sca = g_ref[...] * lax.rsqrt(var + N_EPS)
    poolz = pool_ref[0] + pool_ref[1]
    cgraw = cg_ref[...]
    cg = jnp.maximum(cgraw, 1.0)
    emb = ((poolz / cg - mu) * sca + b_ref[...]) * (cgraw > 0.0).astype(jnp.float32)
    out_ref[...] = emb @ wp_ref[...] + bp_ref[...]


def _tc_d(pools, cg, st2, g2, b2, Wp, bp):
    return pl.pallas_call(
        _tc_d_body,
        out_shape=jax.ShapeDtypeStruct((G, OUT), jnp.float32),
    )(pools, cg, st2, g2.reshape(1, OUT), b2.reshape(1, OUT), Wp, bp.reshape(1, OUT))


def _edge_ticks(ei):
    src = jnp.concatenate([ei[0].astype(jnp.int32), jnp.zeros((EPAD,), jnp.int32)])
    dst = jnp.concatenate([ei[1].astype(jnp.int32), jnp.full((EPAD,), TRASH, jnp.int32)])
    return src.reshape(ETICKS, TICK), dst.reshape(ETICKS, TICK)


def kernel(x_main, x_ctx, edge_index_c2m, edge_index_m2c, batch, Wl1_c2m, bl1_c2m, Wr1_c2m, Wl1_m2c, bl1_m2c, Wr1_m2c, Wl2_c2m, bl2_c2m, Wr2_c2m, Wl2_m2c, bl2_m2c, Wr2_m2c, g1, b1, g2, b2, Wp, bp):
    srcT_m, dstT_m = _edge_ticks(edge_index_c2m)
    srcT_c, dstT_c = _edge_ticks(edge_index_m2c)
    batch_i = batch.astype(jnp.int32)
    batch_pad = jnp.concatenate([batch_i, jnp.full((48,), G, jnp.int32)])

    y1m2, y1c2, r1m2, r1c2 = _tc_prep(
        x_main, x_ctx, Wl1_c2m, Wl1_m2c, Wr1_c2m, Wr1_m2c, bl1_c2m, bl1_m2c)

    agg1m, agg1c, cnts = _sc_l1(y1m2, y1c2, srcT_m, dstT_m, srcT_c, dstT_c)
    cnt_m = cnts[0, :N, 0:1]
    cnt_c = cnts[1, :N, 0:1]

    z1m, st1m = _tc_b1_main(agg1m, r1m2, cnt_m)
    z1c2, st1c4 = _tc_b1_ctx(agg1c, r1c2, cnt_c)
    st1c = jnp.concatenate([st1c4[0], st1c4[1]], axis=1)

    agg2 = _sc_segsum(z1c2, srcT_m, dstT_m)
    z2, st2, cg = _tc_c1(agg2, cnt_m, st1c, g1, b1, Wl2_c2m,
                         z1m, st1m, Wr2_c2m, bl2_c2m, batch_i.reshape(N, 1))

    pools = _sc_pool(z2, batch_pad)
    return _tc_d(pools, cg.reshape(G, 1), st2, g2, b2, Wp, bp)


# speedup vs baseline: 1.2321x; 1.2321x over previous
import functools

import jax
import jax.numpy as jnp
from jax import lax
from jax.experimental import pallas as pl
from jax.experimental.pallas import tpu as pltpu
from jax.experimental.pallas import tpu_sc as plsc

N = 50000
D = 128
H = 64
OUT = 128
G = 512
E = 400000

BS = 2000
NBLK = N // BS

TICK = 128
ETICKS = 3200
EPAD = ETICKS * TICK - E
TRASH = N
ACC_ROWS = N + 48
CNT_ROWS = 3200
SUB_T = ETICKS // 16
SB = 8
N_EPS = 1e-5

_SC_MESH = dict(
    mesh=plsc.VectorSubcoreMesh(core_axis_name="c", subcore_axis_name="s"),
    compiler_params=pltpu.CompilerParams(use_tc_tiling_on_sc=False),
)


def _tc_prep_body(xm, xc, wl1a, wl1b, wr1a, wr1b, bl1a, bl1b,
                  y1m, y1c, r1m, r1c):
    xmv = xm[...]
    xcv = xc[...]
    y1m[...] = xcv @ wl1a[0]
    y1c[...] = xmv @ wl1b[0]
    r1m[...] = xmv @ wr1a[0] + bl1a[0]
    r1c[...] = xcv @ wr1b[0] + bl1b[0]


def _halves(w):
    return w.reshape(w.shape[0], 2, 32).transpose(1, 0, 2)


def _tc_prep(x_main, x_ctx, Wl1_c2m, Wl1_m2c, Wr1_c2m, Wr1_m2c, bl1_c2m, bl1_m2c):
    spec_x = pl.BlockSpec((BS, D), lambda j, i: (i, 0))
    spec_w = pl.BlockSpec((1, D, 32), lambda j, i: (j, 0, 0))
    spec_b = pl.BlockSpec((1, 1, 32), lambda j, i: (j, 0, 0))
    spec_o = pl.BlockSpec((BS, 32), lambda j, i: (j * NBLK + i, 0))
    oshape = jax.ShapeDtypeStruct((2 * N, 32), jnp.float32)
    return pl.pallas_call(
        _tc_prep_body,
        grid=(2, NBLK),
        in_specs=[spec_x, spec_x, spec_w, spec_w, spec_w, spec_w, spec_b, spec_b],
        out_specs=[spec_o, spec_o, spec_o, spec_o],
        out_shape=[oshape, oshape, oshape, oshape],
    )(x_main, x_ctx, _halves(Wl1_c2m), _halves(Wl1_m2c), _halves(Wr1_c2m),
      _halves(Wr1_m2c), bl1_c2m.reshape(2, 1, 32), bl1_m2c.reshape(2, 1, 32))


def _sc_counts_body(dstm, dstc, out, dstbuf, onesb, zbuf, cacc):
    c = lax.axis_index("c")
    s = lax.axis_index("s")
    zeros16 = jnp.zeros((16,), jnp.float32)
    ones16 = jnp.ones((16,), jnp.float32)

    def _zero(r, carry):
        zbuf[r, pl.ds(0, 16)] = zeros16
        return carry
    lax.fori_loop(0, 400, _zero, 0)

    def _ones(r, carry):
        onesb[r, pl.ds(0, 16)] = ones16
        return carry
    lax.fori_loop(0, TICK, _ones, 0)
    for k in range(7):
        pltpu.sync_copy(zbuf, cacc.at[pl.ds(s * 3128 + k * 400, 400)])
    pltpu.sync_copy(zbuf.at[pl.ds(0, 328)], cacc.at[pl.ds(s * 3128 + 2800, 328)])
    plsc.subcore_barrier()

    def _histogram(ref):
        def body(it, carry):
            tb = s * SUB_T + it * SB
            pltpu.sync_copy(ref.at[pl.ds(tb, SB)], dstbuf)
            for r in range(SB):
                pltpu.sync_copy(onesb, cacc.at[dstbuf.at[r]], add=True)
            return carry
        lax.fori_loop(0, SUB_T // SB, body, 0)

    @pl.when(c == 0)
    def _():
        _histogram(dstm)

    @pl.when(c == 1)
    def _():
        _histogram(dstc)

    plsc.subcore_barrier()
    pltpu.sync_copy(cacc.at[pl.ds(s * 3128, 3128)],
                    out.at[c, pl.ds(s * 3128, 3128)])


_sc_counts = functools.partial(
    pl.kernel,
    out_type=jax.ShapeDtypeStruct((2, ACC_ROWS, 16), jnp.float32),
    scratch_types=[
        pltpu.VMEM((SB, TICK), jnp.int32),
        pltpu.VMEM((TICK, 16), jnp.float32),
        pltpu.VMEM((400, 16), jnp.float32),
        pltpu.VMEM_SHARED((ACC_ROWS, 16), jnp.float32),
    ],
    **_SC_MESH,
)(_sc_counts_body)


AROWS = 50048


SSB = 2
NSB = SUB_T // SSB


def _sc_segsum_body(y, src, dst, out,
                    srcb0, dstb0, idxb0, ldst0, rows0,
                    srcb1, dstb1, idxb1, ldst1, rows1,
                    acc, gsem0, gsem1, ssem0, ssem1, isem0, isem1):
    c = lax.axis_index("c")
    s = lax.axis_index("s")
    zeros16 = jnp.zeros((16,), jnp.float32)
    bufs = ((srcb0, dstb0, idxb0, ldst0, rows0, gsem0, ssem0, isem0),
            (srcb1, dstb1, idxb1, ldst1, rows1, gsem1, ssem1, isem1))

    cN = c * N

    def _fire_idx(sb, b):
        srcb, dstb, _, _, _, _, _, isem = bufs[b]
        tb = s * SUB_T + sb * SSB
        pltpu.async_copy(src.at[pl.ds(tb, SSB)], srcb, isem)
        pltpu.async_copy(dst.at[pl.ds(tb, SSB)], dstb, isem)

    def _stage(sb, b, drain_scatter):
        srcb, dstb, idxb, ldstb, rowsb, gsem, ssem, isem = bufs[b]
        tb = s * SUB_T + sb * SSB
        pltpu.make_async_copy(src.at[pl.ds(tb, SSB)], srcb, isem).wait()
        pltpu.make_async_copy(dst.at[pl.ds(tb, SSB)], dstb, isem).wait()
        if drain_scatter:
            for r in range(SSB):
                pltpu.make_async_copy(rowsb.at[pl.ds(r * TICK, TICK)],
                                      acc.at[pl.ds(0, TICK)], ssem).wait()
        for r in range(SSB):
            for t in range(8):
                sl = pl.ds(t * 16, 16)
                idxb[r, sl] = srcb[r, sl] + cN
                ldstb[r, sl] = dstb[r, sl]

        @pl.when(sb + 2 < NSB)
        def _():
            _fire_idx(sb + 2, b)
        handles = []
        for r in range(SSB):
            handles.append(pltpu.async_copy(
                y.at[idxb.at[r]], rowsb.at[pl.ds(r * TICK, TICK)], gsem))
        for h in handles:
            h.wait()
        for r in range(SSB):
            pltpu.async_copy(rowsb.at[pl.ds(r * TICK, TICK)],
                             acc.at[ldstb.at[r]], ssem, add=True)

    def _drain(b):
        _, _, _, _, rowsb, _, ssem, _ = bufs[b]
        for r in range(SSB):
            pltpu.make_async_copy(rowsb.at[pl.ds(r * TICK, TICK)],
                                  acc.at[pl.ds(0, TICK)], ssem).wait()

    def _zero(r, carry):
        rows0[r, pl.ds(0, 16)] = zeros16
        rows0[r, pl.ds(16, 16)] = zeros16
        return carry
    lax.fori_loop(0, SSB * TICK, _zero, 0)
    for k in range(12):
        pltpu.sync_copy(rows0, acc.at[pl.ds(s * 3128 + k * SSB * TICK, SSB * TICK)])
    pltpu.sync_copy(rows0.at[pl.ds(0, 56)],
                    acc.at[pl.ds(s * 3128 + 12 * SSB * TICK, 56)])
    _fire_idx(0, 0)
    _fire_idx(1, 1)
    plsc.subcore_barrier()

    _stage(0, 0, drain_scatter=False)
    _stage(1, 1, drain_scatter=False)

    def body(k, carry):
        _stage(2 + 2 * k, 0, drain_scatter=True)
        _stage(3 + 2 * k, 1, drain_scatter=True)
        return carry
    lax.fori_loop(0, (NSB - 2) // 2, body, 0)
    _drain(0)
    _drain(1)
    plsc.subcore_barrier()

    @pl.when(s < 15)
    def _():
        pltpu.sync_copy(acc.at[pl.ds(s * 3128, 3128)],
                        out.at[pl.ds(c * N + s * 3128, 3128)])

    @pl.when(s == 15)
    def _():
        pltpu.sync_copy(acc.at[pl.ds(15 * 3128, 3080)],
                        out.at[pl.ds(c * N + 15 * 3128, 3080)])


def _seg_bufs():
    return [
        pltpu.VMEM((SSB, TICK), jnp.int32),
        pltpu.VMEM((SSB, TICK), jnp.int32),
        pltpu.VMEM((SSB, TICK), jnp.int32),
        pltpu.VMEM((SSB, TICK), jnp.int32),
        pltpu.VMEM((SSB * TICK, 32), jnp.float32),
    ]


_sc_segsum = functools.partial(
    pl.kernel,
    out_type=jax.ShapeDtypeStruct((2 * N, 32), jnp.float32),
    scratch_types=_seg_bufs() + _seg_bufs() + [
        pltpu.VMEM_SHARED((AROWS, 32), jnp.float32),
        pltpu.SemaphoreType.DMA,
        pltpu.SemaphoreType.DMA,
        pltpu.SemaphoreType.DMA,
        pltpu.SemaphoreType.DMA,
        pltpu.SemaphoreType.DMA,
        pltpu.SemaphoreType.DMA,
    ],
    **_SC_MESH,
)(_sc_segsum_body)


def _sc_pool_body(z2, batch, out, bbuf, btail, rowsb, pacc):
    c = lax.axis_index("c")
    s = lax.axis_index("s")
    w = c * 16 + s
    zeros16 = jnp.zeros((16,), jnp.float32)

    def _zero(r, carry):
        for q in range(8):
            rowsb[r, pl.ds(q * 16, 16)] = zeros16
        return carry
    lax.fori_loop(0, 32, _zero, 0)
    pltpu.sync_copy(rowsb.at[pl.ds(0, 32)], pacc.at[pl.ds(s * 32, 32)])

    @pl.when(s == 0)
    def _():
        pltpu.sync_copy(rowsb.at[pl.ds(0, 32)], pacc.at[pl.ds(G, 32)])
    plsc.subcore_barrier()

    off = 12 * w + jnp.minimum(w, 7)
    n_w = jnp.where(w == 31, 11, jnp.where(w < 7, 13, 12))

    def body(k, carry):
        tk = off + k
        pltpu.sync_copy(z2.at[pl.ds(tk * 128, 128)], rowsb)
        pltpu.sync_copy(batch.at[pl.ds(tk * 128, 128)], bbuf)
        pltpu.sync_copy(rowsb, pacc.at[bbuf], add=True)
        return carry
    lax.fori_loop(0, n_w, body, 0)

    @pl.when(w == 31)
    def _():
        pltpu.sync_copy(z2.at[pl.ds(390 * 128, 80)], rowsb.at[pl.ds(0, 80)])
        pltpu.sync_copy(batch.at[pl.ds(390 * 128, 80)], btail)
        pltpu.sync_copy(rowsb.at[pl.ds(0, 80)], pacc.at[btail], add=True)

    plsc.subcore_barrier()
    pltpu.sync_copy(pacc.at[pl.ds(s * 32, 32)], out.at[c, pl.ds(s * 32, 32)])


_sc_pool = functools.partial(
    pl.kernel,
    out_type=jax.ShapeDtypeStruct((2, G, OUT), jnp.float32),
    scratch_types=[
        pltpu.VMEM((128,), jnp.int32),
        pltpu.VMEM((80,), jnp.int32),
        pltpu.VMEM((128, OUT), jnp.float32),
        pltpu.VMEM_SHARED((G + 32, OUT), jnp.float32),
    ],
    **_SC_MESH,
)(_sc_pool_body)


def _tc_b1_main_body(alo, ahi, rlo, rhi, cnt, z_ref, st_ref):
    i = pl.program_id(0)
    cdiv = jnp.maximum(cnt[...], 1.0)
    a = jnp.concatenate([alo[...], ahi[...]], axis=1) / cdiv
    r = jnp.concatenate([rlo[...], rhi[...]], axis=1)
    z = jnp.maximum(a + r, 0.0)
    z_ref[...] = z

    @pl.when(i == 0)
    def _():
        st_ref[...] = jnp.zeros_like(st_ref)
    st_ref[...] += jnp.stack([jnp.sum(z, axis=0), jnp.sum(z * z, axis=0)])


def _tc_b1_main(agg, r2, cnt):
    return pl.pallas_call(
        _tc_b1_main_body,
        grid=(NBLK,),
        in_specs=[
            pl.BlockSpec((BS, 32), lambda i: (i, 0)),
            pl.BlockSpec((BS, 32), lambda i: (NBLK + i, 0)),
            pl.BlockSpec((BS, 32), lambda i: (i, 0)),
            pl.BlockSpec((BS, 32), lambda i: (NBLK + i, 0)),
            pl.BlockSpec((BS, 1), lambda i: (i, 0)),
        ],
        out_specs=[
            pl.BlockSpec((BS, H), lambda i: (i, 0)),
            pl.BlockSpec((2, H), lambda i: (0, 0)),
        ],
        out_shape=[
            jax.ShapeDtypeStruct((N, H), jnp.float32),
            jax.ShapeDtypeStruct((2, H), jnp.float32),
        ],
    )(agg, agg, r2, r2, cnt)


def _tc_b1_ctx_body(a_ref, r_ref, cnt, z_ref, st_ref):
    i = pl.program_id(1)
    cdiv = jnp.maximum(cnt[...], 1.0)
    z = jnp.maximum(a_ref[...] / cdiv + r_ref[...], 0.0)
    z_ref[...] = z

    @pl.when(i == 0)
    def _():
        st_ref[...] = jnp.zeros_like(st_ref)
    st_ref[...] += jnp.stack([jnp.sum(z, axis=0), jnp.sum(z * z, axis=0)])[None]


def _tc_b1_ctx(agg, r2, cnt):
    spec_half = pl.BlockSpec((BS, 32), lambda j, i: (j * NBLK + i, 0))
    return pl.pallas_call(
        _tc_b1_ctx_body,
        grid=(2, NBLK),
        in_specs=[
            spec_half,
            spec_half,
            pl.BlockSpec((BS, 1), lambda j, i: (i, 0)),
        ],
        out_specs=[
            spec_half,
            pl.BlockSpec((1, 2, 32), lambda j, i: (j, 0, 0)),
        ],
        out_shape=[
            jax.ShapeDtypeStruct((2 * N, 32), jnp.float32),
            jax.ShapeDtypeStruct((2, 2, 32), jnp.float32),
        ],
    )(agg, r2, cnt)


def _tc_c1_body(alo, ahi, cnt, st_ref, g_ref, b_ref, w_ref,
                z1m_ref, stm_ref, wr_ref, bl_ref, batch_ref,
                z2_ref, st2_ref, cg_ref):
    i = pl.program_id(0)
    st = st_ref[...]
    mu = st[0:1, :] * (1.0 / N)
    var = st[1:2, :] * (1.0 / N) - mu * mu
    sca = g_ref[...] * lax.rsqrt(var + N_EPS)
    craw = cnt[...]
    cdiv = jnp.maximum(craw, 1.0)
    a = jnp.concatenate([alo[...], ahi[...]], axis=1) / cdiv
    a2 = ((a - mu) * sca + b_ref[...]) * (craw > 0.0).astype(jnp.float32)
    stm = stm_ref[...]
    mum = stm[0:1, :] * (1.0 / N)
    varm = stm[1:2, :] * (1.0 / N) - mum * mum
    scam = g_ref[...] * lax.rsqrt(varm + N_EPS)
    h = (z1m_ref[...] - mum) * scam + b_ref[...]
    z = jnp.maximum(a2 @ w_ref[...] + h @ wr_ref[...] + bl_ref[...], 0.0)
    z2_ref[...] = z

    @pl.when(i == 0)
    def _():
        st2_ref[...] = jnp.zeros_like(st2_ref)
        cg_ref[...] = jnp.zeros_like(cg_ref)
    st2_ref[...] += jnp.stack([jnp.sum(z, axis=0), jnp.sum(z * z, axis=0)])
    onehot = (batch_ref[...] ==
              lax.broadcasted_iota(jnp.int32, (BS, G), 1)).astype(jnp.float32)
    cg_ref[...] += jnp.sum(onehot, axis=0)[None, :]


def _tc_c1(agg2, cnt, st1c, g1, b1, Wl2, z1m, st1m, Wr2, bl2, batch):
    return pl.pallas_call(
        _tc_c1_body,
        grid=(NBLK,),
        in_specs=[
            pl.BlockSpec((BS, 32), lambda i: (i, 0)),
            pl.BlockSpec((BS, 32), lambda i: (NBLK + i, 0)),
            pl.BlockSpec((BS, 1), lambda i: (i, 0)),
            pl.BlockSpec((2, H), lambda i: (0, 0)),
            pl.BlockSpec((1, H), lambda i: (0, 0)),
            pl.BlockSpec((1, H), lambda i: (0, 0)),
            pl.BlockSpec((H, OUT), lambda i: (0, 0)),
            pl.BlockSpec((BS, H), lambda i: (i, 0)),
            pl.BlockSpec((2, H), lambda i: (0, 0)),
            pl.BlockSpec((H, OUT), lambda i: (0, 0)),
            pl.BlockSpec((1, OUT), lambda i: (0, 0)),
            pl.BlockSpec((BS, 1), lambda i: (i, 0)),
        ],
        out_specs=[
            pl.BlockSpec((BS, OUT), lambda i: (i, 0)),
            pl.BlockSpec((2, OUT), lambda i: (0, 0)),
            pl.BlockSpec((1, G), lambda i: (0, 0)),
        ],
        out_shape=[
            jax.ShapeDtypeStruct((N, OUT), jnp.float32),
            jax.ShapeDtypeStruct((2, OUT), jnp.float32),
            jax.ShapeDtypeStruct((1, G), jnp.float32),
        ],
    )(agg2, agg2, cnt, st1c, g1.reshape(1, H), b1.reshape(1, H), Wl2,
      z1m, st1m, Wr2, bl2.reshape(1, OUT), batch)


def _tc_d_body(pool_ref, cg_ref, st2_ref, g_ref, b_ref, wp_ref, bp_ref, out_ref):
    st = st2_ref[...]
    mu = st[0:1, :] * (1.0 / N)
    var = st[1:2, :] * (1.0 / N) - mu * mu
    sca = g_ref[...] * lax.rsqrt(var + N_EPS)
    poolz = pool_ref[0] + pool_ref[1]
    cgraw = cg_ref[...]
    cg = jnp.maximum(cgraw, 1.0)
    emb = ((poolz / cg - mu) * sca + b_ref[...]) * (cgraw > 0.0).astype(jnp.float32)
    out_ref[...] = emb @ wp_ref[...] + bp_ref[...]


def _tc_d(pools, cg, st2, g2, b2, Wp, bp):
    return pl.pallas_call(
        _tc_d_body,
        out_shape=jax.ShapeDtypeStruct((G, OUT), jnp.float32),
    )(pools, cg, st2, g2.reshape(1, OUT), b2.reshape(1, OUT), Wp, bp.reshape(1, OUT))


def _edge_ticks(ei):
    src = jnp.concatenate([ei[0].astype(jnp.int32), jnp.zeros((EPAD,), jnp.int32)])
    dst = jnp.concatenate([ei[1].astype(jnp.int32), jnp.full((EPAD,), TRASH, jnp.int32)])
    return src.reshape(ETICKS, TICK), dst.reshape(ETICKS, TICK)


_BISECT_JNP_COUNTS = False
_BISECT_JNP_SEGSUM = False
_BISECT_JNP_POOL = False


def _jnp_segsum(y2, ei):
    y = jnp.concatenate([y2[:N], y2[N:]], axis=1)
    s = jax.ops.segment_sum(jnp.take(y, ei[0], axis=0), ei[1], num_segments=N)
    return jnp.concatenate([s[:, :32], s[:, 32:]], axis=0)


def kernel(x_main, x_ctx, edge_index_c2m, edge_index_m2c, batch, Wl1_c2m, bl1_c2m, Wr1_c2m, Wl1_m2c, bl1_m2c, Wr1_m2c, Wl2_c2m, bl2_c2m, Wr2_c2m, Wl2_m2c, bl2_m2c, Wr2_m2c, g1, b1, g2, b2, Wp, bp):
    srcT_m, dstT_m = _edge_ticks(edge_index_c2m)
    srcT_c, dstT_c = _edge_ticks(edge_index_m2c)
    batch_i = batch.astype(jnp.int32)
    batch_pad = jnp.concatenate([batch_i, jnp.full((48,), G, jnp.int32)])

    y1m2, y1c2, r1m2, r1c2 = _tc_prep(
        x_main, x_ctx, Wl1_c2m, Wl1_m2c, Wr1_c2m, Wr1_m2c, bl1_c2m, bl1_m2c)

    if _BISECT_JNP_COUNTS:
        ones_e = jnp.ones((E,), jnp.float32)
        cnt_m = jax.ops.segment_sum(ones_e, edge_index_c2m[1], num_segments=N).reshape(N, 1)
        cnt_c = jax.ops.segment_sum(ones_e, edge_index_m2c[1], num_segments=N).reshape(N, 1)
    else:
        cnts = _sc_counts(dstT_m, dstT_c)
        cnt_m = cnts[0, :N, 0:1]
        cnt_c = cnts[1, :N, 0:1]

    if _BISECT_JNP_SEGSUM:
        agg1m = _jnp_segsum(y1m2, edge_index_c2m)
        agg1c = _jnp_segsum(y1c2, edge_index_m2c)
    else:
        agg1m = _sc_segsum(y1m2, srcT_m, dstT_m)
        agg1c = _sc_segsum(y1c2, srcT_c, dstT_c)

    z1m, st1m = _tc_b1_main(agg1m, r1m2, cnt_m)
    z1c2, st1c4 = _tc_b1_ctx(agg1c, r1c2, cnt_c)
    st1c = jnp.concatenate([st1c4[0], st1c4[1]], axis=1)

    if _BISECT_JNP_SEGSUM:
        agg2 = _jnp_segsum(z1c2, edge_index_c2m)
    else:
        agg2 = _sc_segsum(z1c2, srcT_m, dstT_m)
    z2, st2, cg = _tc_c1(agg2, cnt_m, st1c, g1, b1, Wl2_c2m,
                         z1m, st1m, Wr2_c2m, bl2_c2m, batch_i.reshape(N, 1))

    if _BISECT_JNP_POOL:
        pools = jax.ops.segment_sum(z2, batch_i, num_segments=G)[None]
        pools = jnp.concatenate([pools, jnp.zeros_like(pools)], axis=0)
    else:
        pools = _sc_pool(z2, batch_pad)
    return _tc_d(pools, cg.reshape(G, 1), st2, g2, b2, Wp, bp)
